# baseline (speedup 1.0000x reference)
import functools

import jax
import jax.numpy as jnp
from jax import lax
from jax.experimental import pallas as pl
from jax.experimental.pallas import tpu as pltpu
from jax.experimental.pallas import tpu_sc as plsc

N = 10000
E = 160000
D = 128
H = 128
NC = 64
K = 16
ETA = 0.5
B = 4096

NPAD = 10240
EP = 163840
EPT = EP // 32
NCH = EPT // 128
ZROWS = NPAD // 16

_HIGH = jax.lax.Precision.HIGHEST


def _mesh():
    return plsc.VectorSubcoreMesh(core_axis_name="c", subcore_axis_name="s")


DW = 128


def _sc_deg_body(cols_hbm, ones_hbm, zeros_hbm, out_hbm, colv, onesbuf, dacc,
                 sem):
    cid = lax.axis_index("c")
    sid = lax.axis_index("s")
    wid = cid * 16 + sid
    pltpu.sync_copy(cols_hbm.at[wid], colv)
    pltpu.sync_copy(ones_hbm, onesbuf)
    pltpu.sync_copy(zeros_hbm, dacc.at[pl.ds(sid * ZROWS, ZROWS)])
    plsc.subcore_barrier()

    def body(j, carry):
        pltpu.async_copy(onesbuf, dacc.at[colv.at[j]], sem, add=True)
        return carry

    lax.fori_loop(0, NCH, body, 0)

    def drain(j, carry):
        pltpu.make_async_copy(ones_hbm, onesbuf, sem).wait()
        return carry

    lax.fori_loop(0, NCH, drain, 0)
    plsc.subcore_barrier()
    pltpu.sync_copy(dacc.at[pl.ds(sid * ZROWS, ZROWS)],
                    out_hbm.at[pl.ds(cid * NPAD + sid * ZROWS, ZROWS)])


def _sc_deg(cols3, ones_d, zeros_d):
    f = pl.kernel(
        _sc_deg_body,
        out_type=jax.ShapeDtypeStruct((2 * NPAD, DW), jnp.float32),
        mesh=_mesh(),
        scratch_types=[
            pltpu.VMEM((NCH, 128), jnp.int32),
            pltpu.VMEM((128, DW), jnp.float32),
            pltpu.VMEM_SHARED((NPAD, DW), jnp.float32),
            pltpu.SemaphoreType.DMA,
        ],
    )
    return f(cols3, ones_d, zeros_d)


NBUF = 4
NGRP = NCH // NBUF
ADUM = 256
AROWS = B + ADUM
AZ = AROWS // 16


def _sc_seg_body(rows_hbm, cols_hbm, u_hbm, zeros_hbm, out_hbm,
                 rowv, colv, cmap, b0, b1, b2, b3, accum,
                 g0, g1, g2, g3, s0, s1, s2, s3):
    bufs = (b0, b1, b2, b3)
    gsem = (g0, g1, g2, g3)
    ssem = (s0, s1, s2, s3)
    cid = lax.axis_index("c")
    sid = lax.axis_index("s")
    wid = cid * 16 + sid
    pltpu.sync_copy(rows_hbm.at[wid], rowv)
    pltpu.sync_copy(cols_hbm.at[wid], colv)
    pltpu.sync_copy(zeros_hbm, accum.at[pl.ds(sid * AZ, AZ)])

    for j in range(NCH):
        for k in range(8):
            c = colv[j, pl.ds(k * 16, 16)]
            cmap[j, pl.ds(k * 16, 16)] = jnp.where(
                c < B, c, B + (c & (ADUM - 1)))
    plsc.subcore_barrier()

    dummy = u_hbm.at[pl.ds(0, 128)]

    for i in range(NBUF):
        pltpu.async_copy(u_hbm.at[rowv.at[i]], bufs[i], gsem[i])

    def group(g, carry):
        for i in range(NBUF):
            j = g * NBUF + i
            pltpu.make_async_copy(dummy, bufs[i], gsem[i]).wait()
            pltpu.async_copy(bufs[i], accum.at[cmap.at[j]], ssem[i],
                             add=True)

        @pl.when(g < NGRP - 1)
        def _():
            for i in range(NBUF):
                j = (g + 1) * NBUF + i
                pltpu.make_async_copy(dummy, bufs[i], ssem[i]).wait()
                pltpu.async_copy(u_hbm.at[rowv.at[j]], bufs[i], gsem[i])

        return carry

    lax.fori_loop(0, NGRP, group, 0)
    for i in range(NBUF):
        pltpu.make_async_copy(dummy, bufs[i], ssem[i]).wait()
    plsc.subcore_barrier()
    pltpu.sync_copy(accum.at[pl.ds(sid * (B // 16), B // 16)],
                    out_hbm.at[pl.ds(cid * B + sid * (B // 16), B // 16)])


def _sc_seg(rows3, cols3, u, zrows):
    f = pl.kernel(
        _sc_seg_body,
        out_type=jax.ShapeDtypeStruct((2 * B, H), jnp.float32),
        mesh=_mesh(),
        scratch_types=[
            pltpu.VMEM((NCH, 128), jnp.int32),
            pltpu.VMEM((NCH, 128), jnp.int32),
            pltpu.VMEM((NCH, 128), jnp.int32),
            pltpu.VMEM((128, H), jnp.float32),
            pltpu.VMEM((128, H), jnp.float32),
            pltpu.VMEM((128, H), jnp.float32),
            pltpu.VMEM((128, H), jnp.float32),
            pltpu.VMEM_SHARED((AROWS, H), jnp.float32),
            pltpu.SemaphoreType.DMA,
            pltpu.SemaphoreType.DMA,
            pltpu.SemaphoreType.DMA,
            pltpu.SemaphoreType.DMA,
            pltpu.SemaphoreType.DMA,
            pltpu.SemaphoreType.DMA,
            pltpu.SemaphoreType.DMA,
            pltpu.SemaphoreType.DMA,
        ],
    )
    return f(rows3, cols3, u, zrows)


def _tc1v_body(x_ref, w1_ref, b1_ref, mv_ref, bv_ref, v_ref):
    h = lax.dot_general(x_ref[...].astype(jnp.bfloat16),
                        w1_ref[...].astype(jnp.bfloat16),
                        (((1,), (1,)), ((), ())),
                        preferred_element_type=jnp.float32)
    h = jnp.maximum(h + b1_ref[...], 0.0)
    v = lax.dot_general(h.astype(jnp.bfloat16),
                        mv_ref[...].astype(jnp.bfloat16),
                        (((1,), (0,)), ((), ())),
                        preferred_element_type=jnp.float32)
    v_ref[...] = v + bv_ref[...]


def _tc1v(x, W1, b1, Mv, bv):
    blk = 2048
    grid = (NPAD // blk,)
    return pl.pallas_call(
        _tc1v_body,
        grid=grid,
        in_specs=[
            pl.BlockSpec((blk, D), lambda i: (i, 0)),
            pl.BlockSpec((H, D), lambda i: (0, 0)),
            pl.BlockSpec((1, H), lambda i: (0, 0)),
            pl.BlockSpec((H, H), lambda i: (0, 0)),
            pl.BlockSpec((1, H), lambda i: (0, 0)),
        ],
        out_specs=pl.BlockSpec((blk, H), lambda i: (i, 0)),
        out_shape=jax.ShapeDtypeStruct((N, H), jnp.float32),
    )(x, W1, b1, Mv, bv)


def _tc1u_body(v_ref, d0_ref, d1_ref, u_ref):
    deg = d0_ref[:, 0:1] + d1_ref[:, 0:1]
    dinv = lax.rsqrt(deg + 1.0)
    u_ref[...] = dinv * v_ref[...]


def _tc1u(v, deg2):
    blk = 2048
    grid = (NPAD // blk,)
    return pl.pallas_call(
        _tc1u_body,
        grid=grid,
        in_specs=[
            pl.BlockSpec((blk, H), lambda i: (i, 0)),
            pl.BlockSpec((blk, DW), lambda i: (i, 0)),
            pl.BlockSpec((blk, DW), lambda i: (i + NPAD // blk, 0)),
        ],
        out_specs=pl.BlockSpec((blk, H), lambda i: (i, 0)),
        out_shape=jax.ShapeDtypeStruct((N, H), jnp.float32),
    )(v, deg2, deg2)


def _tc2a_body(p0_ref, p1_ref, u_ref, d0_ref, d1_ref, y_ref, w2_ref, b2_ref,
               emb_ref, en_ref, plc_ref, oh_ref):
    deg = d0_ref[:, 0:1] + d1_ref[:, 0:1]
    dinv = lax.rsqrt(deg + 1.0)
    acc = p0_ref[...] + p1_ref[...] + u_ref[...]
    emb = jnp.maximum(dinv * acc, 0.0)
    emb_ref[...] = emb
    nrm = jnp.sqrt(jnp.sum(emb * emb, axis=1, keepdims=True))
    en_ref[...] = emb / jnp.maximum(nrm, 1e-8)
    lc = lax.dot_general(emb.astype(jnp.bfloat16),
                         w2_ref[...].astype(jnp.bfloat16),
                         (((1,), (1,)), ((), ())),
                         preferred_element_type=jnp.float32)
    lc = lc + b2_ref[...]
    m = jnp.max(lc, axis=1, keepdims=True)
    ls = lc - m
    plc_ref[...] = ls - jnp.log(jnp.sum(jnp.exp(ls), axis=1, keepdims=True))
    cls = lax.broadcasted_iota(jnp.int32, (p0_ref.shape[0], NC), 1)
    oh_ref[...] = jnp.where(y_ref[...] == cls, 1.0, 0.0)


def _tc2a(parts, u, deg2, y2, W2, b2):
    blk = 512
    grid = (B // blk,)
    return pl.pallas_call(
        _tc2a_body,
        grid=grid,
        in_specs=[
            pl.BlockSpec((blk, H), lambda i: (i, 0)),
            pl.BlockSpec((blk, H), lambda i: (i + B // blk, 0)),
            pl.BlockSpec((blk, H), lambda i: (i, 0)),
            pl.BlockSpec((blk, DW), lambda i: (i, 0)),
            pl.BlockSpec((blk, DW), lambda i: (i + NPAD // blk, 0)),
            pl.BlockSpec((blk, 1), lambda i: (i, 0)),
            pl.BlockSpec((NC, H), lambda i: (0, 0)),
            pl.BlockSpec((1, NC), lambda i: (0, 0)),
        ],
        out_specs=[
            pl.BlockSpec((blk, H), lambda i: (i, 0)),
            pl.BlockSpec((blk, H), lambda i: (i, 0)),
            pl.BlockSpec((blk, NC), lambda i: (i, 0)),
            pl.BlockSpec((blk, NC), lambda i: (i, 0)),
        ],
        out_shape=[
            jax.ShapeDtypeStruct((B, H), jnp.float32),
            jax.ShapeDtypeStruct((B, H), jnp.float32),
            jax.ShapeDtypeStruct((B, NC), jnp.float32),
            jax.ShapeDtypeStruct((B, NC), jnp.float32),
        ],
    )(parts, parts, u, deg2, deg2, y2, W2, b2)


def _tc2b_body(en_ref, enf_ref, plc_ref, oh_ref, out_ref):
    s = lax.dot_general(en_ref[...].astype(jnp.bfloat16),
                        enf_ref[...].astype(jnp.bfloat16),
                        (((1,), (1,)), ((), ())),
                        preferred_element_type=jnp.float32)
    w = s
    t = None
    for _ in range(K):
        t = jnp.max(w, axis=1, keepdims=True)
        w = jnp.where(w >= t, -jnp.inf, w)
    gt = (s > t).astype(jnp.float32)
    eq = (s == t).astype(jnp.float32)
    c_gt = jnp.sum(gt, axis=1, keepdims=True)
    c_eq = jnp.sum(eq, axis=1, keepdims=True)
    w_eq = jnp.maximum(K - c_gt, 0.0) / jnp.maximum(c_eq, 1.0)
    mexp = jnp.exp(s) * (gt + eq * w_eq)
    fuse = lax.dot_general(mexp, oh_ref[...], (((1,), (0,)), ((), ())),
                           preferred_element_type=jnp.float32, precision=_HIGH)
    m = jnp.max(fuse, axis=1, keepdims=True)
    ls = fuse - m
    p_sim = ls - jnp.log(jnp.sum(jnp.exp(ls), axis=1, keepdims=True))
    out_ref[...] = ETA * plc_ref[...] + (1.0 - ETA) * p_sim


def _tc2b(en, p_lc, oh):
    blk = 512
    grid = (B // blk,)
    return pl.pallas_call(
        _tc2b_body,
        grid=grid,
        in_specs=[
            pl.BlockSpec((blk, H), lambda i: (i, 0)),
            pl.BlockSpec((B, H), lambda i: (0, 0)),
            pl.BlockSpec((blk, NC), lambda i: (i, 0)),
            pl.BlockSpec((B, NC), lambda i: (0, 0)),
        ],
        out_specs=pl.BlockSpec((blk, NC), lambda i: (i, 0)),
        out_shape=jax.ShapeDtypeStruct((B, NC), jnp.float32),
    )(en, en, p_lc, oh)


def kernel(x, edge_index, y, W1, b1, Wq, bq, Wk, bk, Wv, bv, W2, b2):
    row = edge_index[0]
    col = edge_index[1]
    pad = EP - E
    rowp = jnp.concatenate([row, jnp.zeros((pad,), jnp.int32)])
    padc = (N + (jnp.arange(pad, dtype=jnp.int32) % (NPAD - N)))
    colp = jnp.concatenate([col, padc])
    rows3 = rowp.reshape(32, NCH, 128)
    cols3 = colp.reshape(32, NCH, 128)

    ones_d = jnp.ones((128, DW), jnp.float32)
    zeros_d = jnp.zeros((ZROWS, DW), jnp.float32)
    zrows = jnp.zeros((AZ, H), jnp.float32)

    deg2 = _sc_deg(cols3, ones_d, zeros_d)

    Mv = jax.scipy.linalg.block_diag(*[Wv[g] for g in range(Wv.shape[0])])
    v = _tc1v(x, W1, b1.reshape(1, H), Mv, bv.reshape(1, H))
    u = _tc1u(v, deg2)

    parts = _sc_seg(rows3, cols3, u, zrows)

    y2 = y.reshape(B, 1)
    emb, en, p_lc, oh = _tc2a(parts, u, deg2, y2, W2, b2.reshape(1, NC))

    final = _tc2b(en, p_lc, oh)
    return final, emb


# speedup vs baseline: 1.1402x; 1.1402x over previous
import jax
import jax.numpy as jnp
from jax import lax
from jax.experimental import pallas as pl
from jax.experimental.pallas import tpu as pltpu
from jax.experimental.pallas import tpu_sc as plsc

N = 10000
E = 160000
D = 128
H = 128
NC = 64
K = 16
ETA = 0.5
B = 4096

NPAD = 10240
EP = 163840
EPT = EP // 32
NCH = EPT // 128
ZROWS = NPAD // 16

_HIGH = jax.lax.Precision.HIGHEST


def _mesh():
    return plsc.VectorSubcoreMesh(core_axis_name="c", subcore_axis_name="s")


DW = 128


def _sc_deg_body(cols_hbm, ones_hbm, zeros_hbm, out_hbm, colv, onesbuf, dacc,
                 sem):
    cid = lax.axis_index("c")
    sid = lax.axis_index("s")
    wid = cid * 16 + sid
    pltpu.sync_copy(cols_hbm.at[wid], colv)
    pltpu.sync_copy(ones_hbm, onesbuf)
    pltpu.sync_copy(zeros_hbm, dacc.at[pl.ds(sid * ZROWS, ZROWS)])
    plsc.subcore_barrier()

    def body(j, carry):
        pltpu.async_copy(onesbuf, dacc.at[colv.at[j]], sem, add=True)
        return carry

    lax.fori_loop(0, NCH, body, 0)

    def drain(j, carry):
        pltpu.make_async_copy(ones_hbm, onesbuf, sem).wait()
        return carry

    lax.fori_loop(0, NCH, drain, 0)
    plsc.subcore_barrier()
    pltpu.sync_copy(dacc.at[pl.ds(sid * ZROWS, ZROWS)],
                    out_hbm.at[pl.ds(cid * NPAD + sid * ZROWS, ZROWS)])


def _sc_deg(cols3, ones_d, zeros_d):
    f = pl.kernel(
        _sc_deg_body,
        out_type=jax.ShapeDtypeStruct((2 * NPAD, DW), jnp.float32),
        mesh=_mesh(),
        scratch_types=[
            pltpu.VMEM((NCH, 128), jnp.int32),
            pltpu.VMEM((128, DW), jnp.float32),
            pltpu.VMEM_SHARED((NPAD, DW), jnp.float32),
            pltpu.SemaphoreType.DMA,
        ],
    )
    return f(cols3, ones_d, zeros_d)


NBUF = 4
NGRP = NCH // NBUF
ADUM = 256
AROWS = B + ADUM
AZ = AROWS // 16


def _sc_seg_body(rows_hbm, cols_hbm, u_hbm, zeros_hbm, out_hbm,
                 rowv, colv, cmap, b0, b1, b2, b3, accum,
                 g0, g1, g2, g3, s0, s1, s2, s3):
    bufs = (b0, b1, b2, b3)
    gsem = (g0, g1, g2, g3)
    ssem = (s0, s1, s2, s3)
    cid = lax.axis_index("c")
    sid = lax.axis_index("s")
    wid = cid * 16 + sid
    pltpu.sync_copy(rows_hbm.at[wid], rowv)
    pltpu.sync_copy(cols_hbm.at[wid], colv)
    pltpu.sync_copy(zeros_hbm, accum.at[pl.ds(sid * AZ, AZ)])

    for j in range(NCH):
        for k in range(8):
            c = colv[j, pl.ds(k * 16, 16)]
            cmap[j, pl.ds(k * 16, 16)] = jnp.where(
                c < B, c, B + (c & (ADUM - 1)))
    plsc.subcore_barrier()

    dummy = u_hbm.at[pl.ds(0, 128)]

    for i in range(NBUF):
        pltpu.async_copy(u_hbm.at[rowv.at[i]], bufs[i], gsem[i])

    def group(g, carry):
        for i in range(NBUF):
            j = g * NBUF + i
            pltpu.make_async_copy(dummy, bufs[i], gsem[i]).wait()
            pltpu.async_copy(bufs[i], accum.at[cmap.at[j]], ssem[i],
                             add=True)

        @pl.when(g < NGRP - 1)
        def _():
            for i in range(NBUF):
                j = (g + 1) * NBUF + i
                pltpu.make_async_copy(dummy, bufs[i], ssem[i]).wait()
                pltpu.async_copy(u_hbm.at[rowv.at[j]], bufs[i], gsem[i])

        return carry

    lax.fori_loop(0, NGRP, group, 0)
    for i in range(NBUF):
        pltpu.make_async_copy(dummy, bufs[i], ssem[i]).wait()
    plsc.subcore_barrier()
    pltpu.sync_copy(accum.at[pl.ds(sid * (B // 16), B // 16)],
                    out_hbm.at[pl.ds(cid * B + sid * (B // 16), B // 16)])


def _sc_seg(rows3, cols3, u, zrows):
    f = pl.kernel(
        _sc_seg_body,
        out_type=jax.ShapeDtypeStruct((2 * B, H), jnp.float32),
        mesh=_mesh(),
        scratch_types=[
            pltpu.VMEM((NCH, 128), jnp.int32),
            pltpu.VMEM((NCH, 128), jnp.int32),
            pltpu.VMEM((NCH, 128), jnp.int32),
            pltpu.VMEM((128, H), jnp.float32),
            pltpu.VMEM((128, H), jnp.float32),
            pltpu.VMEM((128, H), jnp.float32),
            pltpu.VMEM((128, H), jnp.float32),
            pltpu.VMEM_SHARED((AROWS, H), jnp.float32),
            pltpu.SemaphoreType.DMA,
            pltpu.SemaphoreType.DMA,
            pltpu.SemaphoreType.DMA,
            pltpu.SemaphoreType.DMA,
            pltpu.SemaphoreType.DMA,
            pltpu.SemaphoreType.DMA,
            pltpu.SemaphoreType.DMA,
            pltpu.SemaphoreType.DMA,
        ],
    )
    return f(rows3, cols3, u, zrows)


def _tc1_body(x_ref, d0_ref, d1_ref, w1_ref, b1_ref, mv_ref, bv_ref, u_ref):
    h = lax.dot_general(x_ref[...].astype(jnp.bfloat16),
                        w1_ref[...].astype(jnp.bfloat16),
                        (((1,), (1,)), ((), ())),
                        preferred_element_type=jnp.float32)
    h = jnp.maximum(h + b1_ref[...], 0.0)
    v = lax.dot_general(h.astype(jnp.bfloat16),
                        mv_ref[...].astype(jnp.bfloat16),
                        (((1,), (0,)), ((), ())),
                        preferred_element_type=jnp.float32)
    v = v + bv_ref[...]
    deg = d0_ref[:, 0:1] + d1_ref[:, 0:1]
    dinv = lax.rsqrt(deg + 1.0)
    u_ref[...] = dinv * v


def _tc1(x, deg2, W1, b1, Mv, bv):
    blk = 2048
    grid = (NPAD // blk,)
    return pl.pallas_call(
        _tc1_body,
        grid=grid,
        in_specs=[
            pl.BlockSpec((blk, D), lambda i: (i, 0)),
            pl.BlockSpec((blk, DW), lambda i: (i, 0)),
            pl.BlockSpec((blk, DW), lambda i: (i + NPAD // blk, 0)),
            pl.BlockSpec((H, D), lambda i: (0, 0)),
            pl.BlockSpec((1, H), lambda i: (0, 0)),
            pl.BlockSpec((H, H), lambda i: (0, 0)),
            pl.BlockSpec((1, H), lambda i: (0, 0)),
        ],
        out_specs=pl.BlockSpec((blk, H), lambda i: (i, 0)),
        out_shape=jax.ShapeDtypeStruct((N, H), jnp.float32),
    )(x, deg2, deg2, W1, b1, Mv, bv)


def _tc2a_body(p0_ref, p1_ref, u_ref, d0_ref, d1_ref, y_ref, w2_ref, b2_ref,
               emb_ref, en_ref, plc_ref, oh_ref):
    deg = d0_ref[:, 0:1] + d1_ref[:, 0:1]
    dinv = lax.rsqrt(deg + 1.0)
    acc = p0_ref[...] + p1_ref[...] + u_ref[...]
    emb = jnp.maximum(dinv * acc, 0.0)
    emb_ref[...] = emb
    nrm = jnp.sqrt(jnp.sum(emb * emb, axis=1, keepdims=True))
    en_ref[...] = emb / jnp.maximum(nrm, 1e-8)
    lc = lax.dot_general(emb.astype(jnp.bfloat16),
                         w2_ref[...].astype(jnp.bfloat16),
                         (((1,), (1,)), ((), ())),
                         preferred_element_type=jnp.float32)
    lc = lc + b2_ref[...]
    m = jnp.max(lc, axis=1, keepdims=True)
    ls = lc - m
    plc_ref[...] = ls - jnp.log(jnp.sum(jnp.exp(ls), axis=1, keepdims=True))
    cls = lax.broadcasted_iota(jnp.int32, (p0_ref.shape[0], NC), 1)
    oh_ref[...] = jnp.where(y_ref[...] == cls, 1.0, 0.0)


def _tc2a(parts, u, deg2, y2, W2, b2):
    blk = 512
    grid = (B // blk,)
    return pl.pallas_call(
        _tc2a_body,
        grid=grid,
        in_specs=[
            pl.BlockSpec((blk, H), lambda i: (i, 0)),
            pl.BlockSpec((blk, H), lambda i: (i + B // blk, 0)),
            pl.BlockSpec((blk, H), lambda i: (i, 0)),
            pl.BlockSpec((blk, DW), lambda i: (i, 0)),
            pl.BlockSpec((blk, DW), lambda i: (i + NPAD // blk, 0)),
            pl.BlockSpec((blk, 1), lambda i: (i, 0)),
            pl.BlockSpec((NC, H), lambda i: (0, 0)),
            pl.BlockSpec((1, NC), lambda i: (0, 0)),
        ],
        out_specs=[
            pl.BlockSpec((blk, H), lambda i: (i, 0)),
            pl.BlockSpec((blk, H), lambda i: (i, 0)),
            pl.BlockSpec((blk, NC), lambda i: (i, 0)),
            pl.BlockSpec((blk, NC), lambda i: (i, 0)),
        ],
        out_shape=[
            jax.ShapeDtypeStruct((B, H), jnp.float32),
            jax.ShapeDtypeStruct((B, H), jnp.float32),
            jax.ShapeDtypeStruct((B, NC), jnp.float32),
            jax.ShapeDtypeStruct((B, NC), jnp.float32),
        ],
    )(parts, parts, u, deg2, deg2, y2, W2, b2)


def _tc2b_body(en_ref, enf_ref, plc_ref, oh_ref, out_ref):
    s = lax.dot_general(en_ref[...].astype(jnp.bfloat16),
                        enf_ref[...].astype(jnp.bfloat16),
                        (((1,), (1,)), ((), ())),
                        preferred_element_type=jnp.float32)
    w = s
    t = None
    for _ in range(K):
        t = jnp.max(w, axis=1, keepdims=True)
        w = jnp.where(w >= t, -jnp.inf, w)
    gt = (s > t).astype(jnp.float32)
    eq = (s == t).astype(jnp.float32)
    c_gt = jnp.sum(gt, axis=1, keepdims=True)
    c_eq = jnp.sum(eq, axis=1, keepdims=True)
    w_eq = jnp.maximum(K - c_gt, 0.0) / jnp.maximum(c_eq, 1.0)
    mexp = jnp.exp(s) * (gt + eq * w_eq)
    fuse = lax.dot_general(mexp, oh_ref[...], (((1,), (0,)), ((), ())),
                           preferred_element_type=jnp.float32, precision=_HIGH)
    m = jnp.max(fuse, axis=1, keepdims=True)
    ls = fuse - m
    p_sim = ls - jnp.log(jnp.sum(jnp.exp(ls), axis=1, keepdims=True))
    out_ref[...] = ETA * plc_ref[...] + (1.0 - ETA) * p_sim


def _tc2b(en, p_lc, oh):
    blk = 512
    grid = (B // blk,)
    return pl.pallas_call(
        _tc2b_body,
        grid=grid,
        in_specs=[
            pl.BlockSpec((blk, H), lambda i: (i, 0)),
            pl.BlockSpec((B, H), lambda i: (0, 0)),
            pl.BlockSpec((blk, NC), lambda i: (i, 0)),
            pl.BlockSpec((B, NC), lambda i: (0, 0)),
        ],
        out_specs=pl.BlockSpec((blk, NC), lambda i: (i, 0)),
        out_shape=jax.ShapeDtypeStruct((B, NC), jnp.float32),
    )(en, en, p_lc, oh)


def kernel(x, edge_index, y, W1, b1, Wq, bq, Wk, bk, Wv, bv, W2, b2):
    row = edge_index[0]
    col = edge_index[1]
    pad = EP - E
    rowp = jnp.concatenate([row, jnp.zeros((pad,), jnp.int32)])
    padc = (N + (jnp.arange(pad, dtype=jnp.int32) % (NPAD - N)))
    colp = jnp.concatenate([col, padc])
    rows3 = rowp.reshape(32, NCH, 128)
    cols3 = colp.reshape(32, NCH, 128)

    ones_d = jnp.ones((128, DW), jnp.float32)
    zeros_d = jnp.zeros((ZROWS, DW), jnp.float32)
    zrows = jnp.zeros((AZ, H), jnp.float32)

    deg2 = _sc_deg(cols3, ones_d, zeros_d)

    Mv = jax.scipy.linalg.block_diag(*[Wv[g] for g in range(Wv.shape[0])])
    u = _tc1(x, deg2, W1, b1.reshape(1, H), Mv, bv.reshape(1, H))

    parts = _sc_seg(rows3, cols3, u, zrows)

    y2 = y.reshape(B, 1)
    emb, en, p_lc, oh = _tc2a(parts, u, deg2, y2, W2, b2.reshape(1, NC))

    final = _tc2b(en, p_lc, oh)
    return final, emb
